# baseline (speedup 1.0000x reference)
import functools

import jax
import jax.numpy as jnp
from jax import lax
from jax.experimental import pallas as pl
from jax.experimental.pallas import tpu as pltpu
from jax.experimental.pallas import tpu_sc as plsc

EMB = 300
NROW = 25
LANES = 16
NPAD = 32
FULL_CHUNKS = 18
TAIL_OFF = EMB - LANES
WPAD = FULL_CHUNKS * LANES + LANES


def _make_sc_call():
    mesh = plsc.VectorSubcoreMesh(core_axis_name="c", subcore_axis_name="s")

    @functools.partial(
        pl.kernel,
        out_type=jax.ShapeDtypeStruct((NPAD,), jnp.float32),
        mesh=mesh,
        compiler_params=pltpu.CompilerParams(
            needs_layout_passes=False, use_tc_tiling_on_sc=False),
        scratch_types=[
            pltpu.VMEM((NPAD,), jnp.int32),
            pltpu.VMEM((WPAD,), jnp.float32),
            pltpu.VMEM((LANES,), jnp.float32),
            pltpu.VMEM((NPAD, EMB), jnp.float32),
            pltpu.VMEM((NPAD, LANES), jnp.float32),
            pltpu.VMEM((NPAD,), jnp.float32),
            pltpu.SemaphoreType.DMA,
        ],
    )
    def sc_fn(idx_hbm, wp_hbm, b_hbm, table_hbm, out_hbm,
              idx_v, w_v, b_v, rows_v, acc_v, out_v, sem):
        cid = lax.axis_index("c")
        sid = lax.axis_index("s")

        @pl.when(jnp.logical_and(cid == 0, sid == 0))
        def _():
            pltpu.sync_copy(idx_hbm, idx_v)
            pltpu.sync_copy(wp_hbm, w_v)
            pltpu.sync_copy(b_hbm, b_v)
            iv0 = idx_v[pl.ds(0, LANES)]
            iv1 = idx_v[pl.ds(LANES, LANES)]
            copies = []
            for i in range(NROW):
                r = iv0[i] if i < LANES else iv1[i - LANES]
                copies.append(pltpu.async_copy(
                    table_hbm.at[pl.ds(r, 1), :],
                    rows_v.at[pl.ds(i, 1), :], sem))
            for c in copies:
                c.wait()

            wchunks = [w_v[pl.ds(c * LANES, LANES)] for c in range(FULL_CHUNKS)]
            wtail = w_v[pl.ds(FULL_CHUNKS * LANES, LANES)]

            lane = lax.iota(jnp.int32, LANES)
            zeros = jnp.zeros((LANES,), jnp.float32)
            for i in range(NROW, NPAD):
                acc_v[i, pl.ds(0, LANES)] = zeros
            for i in range(NROW):
                acc = rows_v[i, pl.ds(0, LANES)] * wchunks[0]
                for c in range(1, FULL_CHUNKS):
                    acc = acc + rows_v[i, pl.ds(c * LANES, LANES)] * wchunks[c]
                acc = acc + rows_v[i, pl.ds(TAIL_OFF, LANES)] * wtail
                acc_v[i, pl.ds(0, LANES)] = acc

            bias = b_v[...]
            for h in range(NPAD // LANES):
                rows_idx = lane + (h * LANES)
                tot = plsc.load_gather(
                    acc_v, [rows_idx, jnp.zeros((LANES,), jnp.int32)])
                for j in range(1, LANES):
                    tot = tot + plsc.load_gather(
                        acc_v, [rows_idx, jnp.full((LANES,), j, jnp.int32)])
                x = tot + bias
                out_v[pl.ds(h * LANES, LANES)] = 1.0 / (1.0 + jnp.exp(-x))

            pltpu.sync_copy(out_v, out_hbm)

    return sc_fn


_SC_CALL = _make_sc_call()


def kernel(input_words, table, W, b):
    idx = jnp.zeros((NPAD,), jnp.int32).at[:NROW].set(input_words[:, -1])
    w0 = W[:, 0]
    wp = jnp.concatenate(
        [w0[: FULL_CHUNKS * LANES], jnp.zeros((4,), jnp.float32), w0[FULL_CHUNKS * LANES:]]
    )
    bvec = jnp.full((LANES,), b[0], jnp.float32)
    out = _SC_CALL(idx, wp, bvec, table)
    return out[:NROW]


# speedup vs baseline: 5.1925x; 5.1925x over previous
import functools

import jax
import jax.numpy as jnp
from jax import lax
from jax.experimental import pallas as pl
from jax.experimental.pallas import tpu as pltpu
from jax.experimental.pallas import tpu_sc as plsc

EMB = 300
NROW = 25
LANES = 16
NPAD = 32
FULL_CHUNKS = 18
TAIL_OFF = EMB - LANES
WPAD = FULL_CHUNKS * LANES + LANES


def _make_sc_call():
    mesh = plsc.VectorSubcoreMesh(core_axis_name="c", subcore_axis_name="s")

    @functools.partial(
        pl.kernel,
        out_type=jax.ShapeDtypeStruct((NPAD,), jnp.float32),
        mesh=mesh,
        compiler_params=pltpu.CompilerParams(
            needs_layout_passes=False, use_tc_tiling_on_sc=True),
        scratch_types=[
            pltpu.VMEM((NPAD,), jnp.int32),
            pltpu.VMEM((WPAD,), jnp.float32),
            pltpu.VMEM((LANES,), jnp.float32),
            pltpu.VMEM((NPAD, EMB), jnp.float32),
            pltpu.VMEM((NPAD, LANES), jnp.float32),
            pltpu.VMEM((NPAD,), jnp.float32),
            pltpu.SemaphoreType.DMA,
        ],
    )
    def sc_fn(idx_hbm, wp_hbm, b_hbm, table_hbm, out_hbm,
              idx_v, w_v, b_v, rows_v, acc_v, out_v, sem):
        cid = lax.axis_index("c")
        sid = lax.axis_index("s")

        @pl.when(jnp.logical_and(cid == 0, sid == 0))
        def _():
            pltpu.sync_copy(idx_hbm, idx_v)
            pltpu.sync_copy(wp_hbm, w_v)
            pltpu.sync_copy(b_hbm, b_v)
            iv0 = idx_v[pl.ds(0, LANES)]
            iv1 = idx_v[pl.ds(LANES, LANES)]
            copies = []
            for i in range(NROW):
                r = iv0[i] if i < LANES else iv1[i - LANES]
                copies.append(pltpu.async_copy(
                    table_hbm.at[pl.ds(r, 1), :],
                    rows_v.at[pl.ds(i, 1), :], sem))
            for c in copies:
                c.wait()

            wchunks = [w_v[pl.ds(c * LANES, LANES)] for c in range(FULL_CHUNKS)]
            wtail = w_v[pl.ds(FULL_CHUNKS * LANES, LANES)]

            lane = lax.iota(jnp.int32, LANES)
            zeros = jnp.zeros((LANES,), jnp.float32)
            for i in range(NROW, NPAD):
                acc_v[i, pl.ds(0, LANES)] = zeros
            for i in range(NROW):
                acc = rows_v[i, pl.ds(0, LANES)] * wchunks[0]
                for c in range(1, FULL_CHUNKS):
                    acc = acc + rows_v[i, pl.ds(c * LANES, LANES)] * wchunks[c]
                acc = acc + rows_v[i, pl.ds(TAIL_OFF, LANES)] * wtail
                acc_v[i, pl.ds(0, LANES)] = acc

            bias = b_v[...]
            for h in range(NPAD // LANES):
                rows_idx = lane + (h * LANES)
                tot = plsc.load_gather(
                    acc_v, [rows_idx, jnp.zeros((LANES,), jnp.int32)])
                for j in range(1, LANES):
                    tot = tot + plsc.load_gather(
                        acc_v, [rows_idx, jnp.full((LANES,), j, jnp.int32)])
                x = tot + bias
                out_v[pl.ds(h * LANES, LANES)] = 1.0 / (1.0 + jnp.exp(-x))

            pltpu.sync_copy(out_v, out_hbm)

    return sc_fn


_SC_CALL = _make_sc_call()


def kernel(input_words, table, W, b):
    idx = jnp.zeros((NPAD,), jnp.int32).at[:NROW].set(input_words[:, -1])
    w0 = W[:, 0]
    wp = jnp.concatenate(
        [w0[: FULL_CHUNKS * LANES], jnp.zeros((4,), jnp.float32), w0[FULL_CHUNKS * LANES:]]
    )
    bvec = jnp.full((LANES,), b[0], jnp.float32)
    out = _SC_CALL(idx, wp, bvec, table)
    return out[:NROW]
